# baseline (speedup 1.0000x reference)
import functools

import jax
import jax.numpy as jnp
from jax import lax
from jax.experimental import pallas as pl
from jax.experimental.pallas import tpu as pltpu
from jax.experimental.pallas import tpu_sc as plsc

NUM_NODES = 10000
NUM_EDGES = 320000

NC = 2
NS = 16
NW = NC * NS
L = 16

EPW = NUM_EDGES // NW
HIST_SLICE = 2 * NUM_EDGES // NS
HBINS = 10240
CHUNK = HBINS // NS


def _tile_body(edge_hbm, out_hbm, idx_v, hist_v, red_v, gch_v, out_v,
               lhist_sp, ghist_sp):
    c = lax.axis_index("c")
    s = lax.axis_index("s")
    wid = s * NC + c
    row = s // 8
    col = (s % 8) * HIST_SLICE

    zeros = jnp.zeros((L,), jnp.int32)
    ones = jnp.full((L,), 1, jnp.int32)

    def _zero(j, carry):
        for u in range(8):
            hist_v[pl.ds(j * (8 * L) + u * L, L)] = zeros
        return carry

    lax.fori_loop(0, HBINS // (8 * L), _zero, 0)

    pltpu.sync_copy(edge_hbm.at[row, pl.ds(col, HIST_SLICE)], idx_v)

    def _hist(i, carry):
        for u in range(4):
            v = idx_v[pl.ds(i * (4 * L) + u * L, L)]
            plsc.addupdate_scatter(hist_v, [v], ones)
        return carry

    lax.fori_loop(0, HIST_SLICE // (4 * L), _hist, 0)

    pltpu.sync_copy(hist_v, lhist_sp.at[s])
    plsc.subcore_barrier()

    for t in range(NS):
        pltpu.sync_copy(lhist_sp.at[t, pl.ds(s * CHUNK, CHUNK)],
                        red_v.at[pl.ds(t * CHUNK, CHUNK)])

    def _reduce(k, carry):
        acc = zeros
        for t in range(NS):
            acc = acc + red_v[pl.ds(t * CHUNK + k * L, L)]
        gch_v[pl.ds(k * L, L)] = acc
        return carry

    lax.fori_loop(0, CHUNK // L, _reduce, 0)

    pltpu.sync_copy(gch_v, ghist_sp.at[pl.ds(s * CHUNK, CHUNK)])
    plsc.subcore_barrier()

    pltpu.sync_copy(ghist_sp, hist_v)

    base = wid * EPW
    pltpu.sync_copy(edge_hbm.at[0, pl.ds(base, EPW)], idx_v.at[pl.ds(0, EPW)])
    pltpu.sync_copy(edge_hbm.at[1, pl.ds(base, EPW)],
                    idx_v.at[pl.ds(EPW, EPW)])

    half = jnp.full((L,), 0.5, jnp.float32)
    threehalf = jnp.full((L,), 1.5, jnp.float32)
    magic = jnp.full((L,), 0x5F3759DF, jnp.int32)

    def _gather(i, carry):
        for u in range(5):
            off = i * (5 * L) + u * L
            sv = idx_v[pl.ds(off, L)]
            dv = idx_v[pl.ds(EPW + off, L)]
            cs = plsc.load_gather(hist_v, [sv])
            cd = plsc.load_gather(hist_v, [dv])
            x = (cs + cd).astype(jnp.float32)
            yi = magic - (plsc.bitcast(x, jnp.int32) >> 1)
            y = plsc.bitcast(yi, jnp.float32)
            for _ in range(2):
                y = y * (threehalf - half * x * y * y)
            out_v[pl.ds(off, L)] = y
        return carry

    lax.fori_loop(0, EPW // (5 * L), _gather, 0)

    pltpu.sync_copy(out_v, out_hbm.at[pl.ds(base, EPW)])


_edge_weight = functools.partial(
    pl.kernel,
    out_type=jax.ShapeDtypeStruct((NUM_EDGES,), jnp.float32),
    mesh=plsc.VectorSubcoreMesh(core_axis_name="c", subcore_axis_name="s"),
    compiler_params=pltpu.CompilerParams(needs_layout_passes=False,
                                         use_tc_tiling_on_sc=False),
    scratch_types=[
        pltpu.VMEM((HIST_SLICE,), jnp.int32),
        pltpu.VMEM((HBINS,), jnp.int32),
        pltpu.VMEM((HBINS,), jnp.int32),
        pltpu.VMEM((CHUNK,), jnp.int32),
        pltpu.VMEM((EPW,), jnp.float32),
        pltpu.VMEM_SHARED((NS, HBINS), jnp.int32),
        pltpu.VMEM_SHARED((HBINS,), jnp.int32),
    ],
)(_tile_body)


NUM_COPY_DMAS = 4
COPY_ROWS = NUM_EDGES // NUM_COPY_DMAS


def _copy_body(m_ref, o_ref, sems):
    copies = [
        pltpu.make_async_copy(
            m_ref.at[pl.ds(i * COPY_ROWS, COPY_ROWS)],
            o_ref.at[pl.ds(i * COPY_ROWS, COPY_ROWS)],
            sems.at[i],
        )
        for i in range(NUM_COPY_DMAS)
    ]
    for cp in copies:
        cp.start()
    for cp in copies:
        cp.wait()


_msg_copy = pl.pallas_call(
    _copy_body,
    in_specs=[pl.BlockSpec(memory_space=pl.ANY)],
    out_specs=pl.BlockSpec(memory_space=pl.ANY),
    out_shape=jax.ShapeDtypeStruct((NUM_EDGES, 128), jnp.float32),
    scratch_shapes=[pltpu.SemaphoreType.DMA((NUM_COPY_DMAS,))],
)


def kernel(edge_index, message, x_e):
    edge_weight = _edge_weight(edge_index)
    message_out = _msg_copy(message)
    return message_out, edge_weight


# speedup vs baseline: 37.7646x; 37.7646x over previous
import functools

import jax
import jax.numpy as jnp
from jax import lax
from jax.experimental import pallas as pl
from jax.experimental.pallas import tpu as pltpu
from jax.experimental.pallas import tpu_sc as plsc

NUM_NODES = 10000
NUM_EDGES = 320000

NC = 2
NS = 16
NW = NC * NS
L = 16

EPW = NUM_EDGES // NW
HIST_SLICE = 2 * NUM_EDGES // NS
HBINS = 10240
CHUNK = HBINS // NS


def _tile_body(edge_hbm, out_hbm, idx_v, hist_v, red_v, gch_v, out_v,
               lhist_sp, ghist_sp):
    c = lax.axis_index("c")
    s = lax.axis_index("s")
    wid = s * NC + c
    row = s // 8
    col = (s % 8) * HIST_SLICE

    zeros = jnp.zeros((L,), jnp.int32)
    ones = jnp.full((L,), 1, jnp.int32)

    def _zero(j, carry):
        for u in range(8):
            hist_v[pl.ds(j * (8 * L) + u * L, L)] = zeros
        return carry

    lax.fori_loop(0, HBINS // (8 * L), _zero, 0)

    pltpu.sync_copy(edge_hbm.at[row, pl.ds(col, HIST_SLICE)], idx_v)

    def _hist(i, carry):
        for u in range(4):
            v = idx_v[pl.ds(i * (4 * L) + u * L, L)]
            plsc.addupdate_scatter(hist_v, [v], ones)
        return carry

    lax.fori_loop(0, HIST_SLICE // (4 * L), _hist, 0)

    pltpu.sync_copy(hist_v, lhist_sp.at[s])
    plsc.subcore_barrier()

    for t in range(NS):
        pltpu.sync_copy(lhist_sp.at[t, pl.ds(s * CHUNK, CHUNK)],
                        red_v.at[pl.ds(t * CHUNK, CHUNK)])

    def _reduce(k, carry):
        acc = zeros
        for t in range(NS):
            acc = acc + red_v[pl.ds(t * CHUNK + k * L, L)]
        gch_v[pl.ds(k * L, L)] = acc
        return carry

    lax.fori_loop(0, CHUNK // L, _reduce, 0)

    pltpu.sync_copy(gch_v, ghist_sp.at[pl.ds(s * CHUNK, CHUNK)])
    plsc.subcore_barrier()

    pltpu.sync_copy(ghist_sp, hist_v)

    base = wid * EPW
    pltpu.sync_copy(edge_hbm.at[0, pl.ds(base, EPW)], idx_v.at[pl.ds(0, EPW)])
    pltpu.sync_copy(edge_hbm.at[1, pl.ds(base, EPW)],
                    idx_v.at[pl.ds(EPW, EPW)])

    half = jnp.full((L,), 0.5, jnp.float32)
    threehalf = jnp.full((L,), 1.5, jnp.float32)
    magic = jnp.full((L,), 0x5F3759DF, jnp.int32)

    def _gather(i, carry):
        for u in range(5):
            off = i * (5 * L) + u * L
            sv = idx_v[pl.ds(off, L)]
            dv = idx_v[pl.ds(EPW + off, L)]
            cs = plsc.load_gather(hist_v, [sv])
            cd = plsc.load_gather(hist_v, [dv])
            x = (cs + cd).astype(jnp.float32)
            yi = magic - (plsc.bitcast(x, jnp.int32) >> 1)
            y = plsc.bitcast(yi, jnp.float32)
            for _ in range(2):
                y = y * (threehalf - half * x * y * y)
            out_v[pl.ds(off, L)] = y
        return carry

    lax.fori_loop(0, EPW // (5 * L), _gather, 0)

    pltpu.sync_copy(out_v, out_hbm.at[pl.ds(base, EPW)])


_edge_weight = functools.partial(
    pl.kernel,
    out_type=jax.ShapeDtypeStruct((NUM_EDGES,), jnp.float32),
    mesh=plsc.VectorSubcoreMesh(core_axis_name="c", subcore_axis_name="s"),
    compiler_params=pltpu.CompilerParams(needs_layout_passes=False,
                                         use_tc_tiling_on_sc=False),
    scratch_types=[
        pltpu.VMEM((HIST_SLICE,), jnp.int32),
        pltpu.VMEM((HBINS,), jnp.int32),
        pltpu.VMEM((HBINS,), jnp.int32),
        pltpu.VMEM((CHUNK,), jnp.int32),
        pltpu.VMEM((EPW,), jnp.float32),
        pltpu.VMEM_SHARED((NS, HBINS), jnp.int32),
        pltpu.VMEM_SHARED((HBINS,), jnp.int32),
    ],
)(_tile_body)


MSG_BLOCK = 5000


def _copy_body(m_ref, o_ref):
    o_ref[...] = m_ref[...]


_msg_copy = pl.pallas_call(
    _copy_body,
    grid=(NUM_EDGES // MSG_BLOCK,),
    in_specs=[pl.BlockSpec((MSG_BLOCK, 128), lambda i: (i, 0))],
    out_specs=pl.BlockSpec((MSG_BLOCK, 128), lambda i: (i, 0)),
    out_shape=jax.ShapeDtypeStruct((NUM_EDGES, 128), jnp.float32),
)


def kernel(edge_index, message, x_e):
    edge_weight = _edge_weight(edge_index)
    message_out = _msg_copy(message)
    return message_out, edge_weight


# speedup vs baseline: 39.9303x; 1.0573x over previous
import functools

import jax
import jax.numpy as jnp
from jax import lax
from jax.experimental import pallas as pl
from jax.experimental.pallas import tpu as pltpu
from jax.experimental.pallas import tpu_sc as plsc

NUM_NODES = 10000
NUM_EDGES = 320000

NC = 2
NS = 16
NW = NC * NS
L = 16

EPW = NUM_EDGES // NW
HIST_SLICE = 2 * NUM_EDGES // NS
HBINS = 10240
CHUNK = HBINS // NS


def _tile_body(edge_hbm, out_hbm, idx_v, hist_v, red_v, gch_v, out_v,
               lhist_sp, ghist_sp):
    c = lax.axis_index("c")
    s = lax.axis_index("s")
    wid = s * NC + c
    row = s // 8
    col = (s % 8) * HIST_SLICE

    zeros = jnp.zeros((L,), jnp.int32)
    ones = jnp.full((L,), 1, jnp.int32)

    def _zero(j, carry):
        for u in range(8):
            hist_v[pl.ds(j * (8 * L) + u * L, L)] = zeros
        return carry

    lax.fori_loop(0, HBINS // (8 * L), _zero, 0)

    pltpu.sync_copy(edge_hbm.at[row, pl.ds(col, HIST_SLICE)], idx_v)

    def _hist(i, carry):
        for u in range(4):
            v = idx_v[pl.ds(i * (4 * L) + u * L, L)]
            plsc.addupdate_scatter(hist_v, [v], ones)
        return carry

    lax.fori_loop(0, HIST_SLICE // (4 * L), _hist, 0)

    pltpu.sync_copy(hist_v, lhist_sp.at[s])
    plsc.subcore_barrier()

    for t in range(NS):
        pltpu.sync_copy(lhist_sp.at[t, pl.ds(s * CHUNK, CHUNK)],
                        red_v.at[pl.ds(t * CHUNK, CHUNK)])

    def _reduce(k, carry):
        acc = zeros
        for t in range(NS):
            acc = acc + red_v[pl.ds(t * CHUNK + k * L, L)]
        gch_v[pl.ds(k * L, L)] = acc
        return carry

    lax.fori_loop(0, CHUNK // L, _reduce, 0)

    pltpu.sync_copy(gch_v, ghist_sp.at[pl.ds(s * CHUNK, CHUNK)])
    plsc.subcore_barrier()

    pltpu.sync_copy(ghist_sp, hist_v)

    base = wid * EPW
    pltpu.sync_copy(edge_hbm.at[0, pl.ds(base, EPW)], idx_v.at[pl.ds(0, EPW)])
    pltpu.sync_copy(edge_hbm.at[1, pl.ds(base, EPW)],
                    idx_v.at[pl.ds(EPW, EPW)])

    half = jnp.full((L,), 0.5, jnp.float32)
    threehalf = jnp.full((L,), 1.5, jnp.float32)
    magic = jnp.full((L,), 0x5F3759DF, jnp.int32)

    def _gather(i, carry):
        for u in range(5):
            off = i * (5 * L) + u * L
            sv = idx_v[pl.ds(off, L)]
            dv = idx_v[pl.ds(EPW + off, L)]
            cs = plsc.load_gather(hist_v, [sv])
            cd = plsc.load_gather(hist_v, [dv])
            x = (cs + cd).astype(jnp.float32)
            yi = magic - (plsc.bitcast(x, jnp.int32) >> 1)
            y = plsc.bitcast(yi, jnp.float32)
            for _ in range(2):
                y = y * (threehalf - half * x * y * y)
            out_v[pl.ds(off, L)] = y
        return carry

    lax.fori_loop(0, EPW // (5 * L), _gather, 0)

    pltpu.sync_copy(out_v, out_hbm.at[pl.ds(base, EPW)])


_edge_weight = functools.partial(
    pl.kernel,
    out_type=jax.ShapeDtypeStruct((NUM_EDGES,), jnp.float32),
    mesh=plsc.VectorSubcoreMesh(core_axis_name="c", subcore_axis_name="s"),
    compiler_params=pltpu.CompilerParams(needs_layout_passes=False,
                                         use_tc_tiling_on_sc=False),
    scratch_types=[
        pltpu.VMEM((HIST_SLICE,), jnp.int32),
        pltpu.VMEM((HBINS,), jnp.int32),
        pltpu.VMEM((HBINS,), jnp.int32),
        pltpu.VMEM((CHUNK,), jnp.int32),
        pltpu.VMEM((EPW,), jnp.float32),
        pltpu.VMEM_SHARED((NS, HBINS), jnp.int32),
        pltpu.VMEM_SHARED((HBINS,), jnp.int32),
    ],
)(_tile_body)


MSG_BLOCK = 20000


def _copy_body(m_ref, o_ref):
    o_ref[...] = m_ref[...]


_msg_copy = pl.pallas_call(
    _copy_body,
    grid=(NUM_EDGES // MSG_BLOCK,),
    in_specs=[pl.BlockSpec((MSG_BLOCK, 128), lambda i: (i, 0))],
    out_specs=pl.BlockSpec((MSG_BLOCK, 128), lambda i: (i, 0)),
    out_shape=jax.ShapeDtypeStruct((NUM_EDGES, 128), jnp.float32),
)


def kernel(edge_index, message, x_e):
    edge_weight = _edge_weight(edge_index)
    message_out = _msg_copy(message)
    return message_out, edge_weight


# speedup vs baseline: 39.9676x; 1.0009x over previous
import functools

import jax
import jax.numpy as jnp
from jax import lax
from jax.experimental import pallas as pl
from jax.experimental.pallas import tpu as pltpu
from jax.experimental.pallas import tpu_sc as plsc

NUM_NODES = 10000
NUM_EDGES = 320000

NC = 2
NS = 16
NW = NC * NS
L = 16

EPW = NUM_EDGES // NW
HIST_SLICE = 2 * NUM_EDGES // NS
HBINS = 10240
CHUNK = HBINS // NS


def _tile_body(edge_hbm, out_hbm, idx_v, hist_v, red_v, gch_v, out_v,
               lhist_sp, ghist_sp):
    c = lax.axis_index("c")
    s = lax.axis_index("s")
    wid = s * NC + c
    row = s // 8
    col = (s % 8) * HIST_SLICE

    zeros = jnp.zeros((L,), jnp.int32)
    ones = jnp.full((L,), 1, jnp.int32)

    def _zero(j, carry):
        for u in range(8):
            hist_v[pl.ds(j * (8 * L) + u * L, L)] = zeros
        return carry

    lax.fori_loop(0, HBINS // (8 * L), _zero, 0)

    pltpu.sync_copy(edge_hbm.at[row, pl.ds(col, HIST_SLICE)], idx_v)

    def _hist(i, carry):
        for u in range(4):
            v = idx_v[pl.ds(i * (4 * L) + u * L, L)]
            plsc.addupdate_scatter(hist_v, [v], ones)
        return carry

    lax.fori_loop(0, HIST_SLICE // (4 * L), _hist, 0)

    pltpu.sync_copy(hist_v, lhist_sp.at[s])
    plsc.subcore_barrier()

    for t in range(NS):
        pltpu.sync_copy(lhist_sp.at[t, pl.ds(s * CHUNK, CHUNK)],
                        red_v.at[pl.ds(t * CHUNK, CHUNK)])

    def _reduce(k, carry):
        acc = zeros
        for t in range(NS):
            acc = acc + red_v[pl.ds(t * CHUNK + k * L, L)]
        gch_v[pl.ds(k * L, L)] = acc
        return carry

    lax.fori_loop(0, CHUNK // L, _reduce, 0)

    pltpu.sync_copy(gch_v, ghist_sp.at[pl.ds(s * CHUNK, CHUNK)])
    plsc.subcore_barrier()

    pltpu.sync_copy(ghist_sp, hist_v)

    base = wid * EPW
    pltpu.sync_copy(edge_hbm.at[0, pl.ds(base, EPW)], idx_v.at[pl.ds(0, EPW)])
    pltpu.sync_copy(edge_hbm.at[1, pl.ds(base, EPW)],
                    idx_v.at[pl.ds(EPW, EPW)])

    half = jnp.full((L,), 0.5, jnp.float32)
    threehalf = jnp.full((L,), 1.5, jnp.float32)
    magic = jnp.full((L,), 0x5F3759DF, jnp.int32)

    def _gather(i, carry):
        for u in range(5):
            off = i * (5 * L) + u * L
            sv = idx_v[pl.ds(off, L)]
            dv = idx_v[pl.ds(EPW + off, L)]
            cs = plsc.load_gather(hist_v, [sv])
            cd = plsc.load_gather(hist_v, [dv])
            x = (cs + cd).astype(jnp.float32)
            yi = magic - (plsc.bitcast(x, jnp.int32) >> 1)
            y = plsc.bitcast(yi, jnp.float32)
            for _ in range(2):
                y = y * (threehalf - half * x * y * y)
            out_v[pl.ds(off, L)] = y
        return carry

    lax.fori_loop(0, EPW // (5 * L), _gather, 0)

    pltpu.sync_copy(out_v, out_hbm.at[pl.ds(base, EPW)])


_edge_weight = functools.partial(
    pl.kernel,
    out_type=jax.ShapeDtypeStruct((NUM_EDGES,), jnp.float32),
    mesh=plsc.VectorSubcoreMesh(core_axis_name="c", subcore_axis_name="s"),
    compiler_params=pltpu.CompilerParams(needs_layout_passes=False,
                                         use_tc_tiling_on_sc=False),
    scratch_types=[
        pltpu.VMEM((HIST_SLICE,), jnp.int32),
        pltpu.VMEM((HBINS,), jnp.int32),
        pltpu.VMEM((HBINS,), jnp.int32),
        pltpu.VMEM((CHUNK,), jnp.int32),
        pltpu.VMEM((EPW,), jnp.float32),
        pltpu.VMEM_SHARED((NS, HBINS), jnp.int32),
        pltpu.VMEM_SHARED((HBINS,), jnp.int32),
    ],
)(_tile_body)


MSG_BLOCK = 16000


def _copy_body(m_ref, o_ref):
    o_ref[...] = m_ref[...]


_msg_copy = pl.pallas_call(
    _copy_body,
    grid=(NUM_EDGES // MSG_BLOCK,),
    in_specs=[pl.BlockSpec((MSG_BLOCK, 128), lambda i: (i, 0))],
    out_specs=pl.BlockSpec((MSG_BLOCK, 128), lambda i: (i, 0)),
    out_shape=jax.ShapeDtypeStruct((NUM_EDGES, 128), jnp.float32),
)


def kernel(edge_index, message, x_e):
    edge_weight = _edge_weight(edge_index)
    message_out = _msg_copy(message)
    return message_out, edge_weight
